# baseline (speedup 1.0000x reference)
import jax
import jax.numpy as jnp
from jax import lax
from jax.experimental import pallas as pl
from jax.experimental.pallas import tpu as pltpu
from jax.experimental.pallas import tpu_sc as plsc

B = 16384
K = 20
HID = 64
VOCAB = 1000000

NC = 2
NS = 16
NW = NC * NS
BPW = B // NW
CHUNK = 32
NCHUNK = BPW // CHUNK
CROWS = CHUNK * K
NSUB = CHUNK // 16


def _sc_dots_body(wemb_hbm, cemb_hbm, widx_hbm, cidx_hbm, out_hbm,
                  widx_v, cidx_v, widx2_v, cidx2_v, wrows_v, crows_v, res_v,
                  sem0, sem1, sem2):
    wid = lax.axis_index("s") * NC + lax.axis_index("c")
    lanes = lax.iota(jnp.int32, 16)

    def chunk_body(i, _):
        base = wid * BPW + i * CHUNK
        pltpu.sync_copy(widx_hbm.at[pl.ds(base, CHUNK)], widx_v)
        idx_cps = [
            pltpu.async_copy(cidx_hbm.at[k, pl.ds(base, CHUNK)],
                             cidx_v.at[pl.ds(k * CHUNK, CHUNK)], sem2)
            for k in range(K)]
        for cp in idx_cps:
            cp.wait()
        for j in range(CHUNK // 16):
            widx2_v[pl.ds(j * 16, 16)] = lax.shift_right_logical(
                widx_v[pl.ds(j * 16, 16)], 1)
        for j in range(CROWS // 16):
            cidx2_v[pl.ds(j * 16, 16)] = lax.shift_right_logical(
                cidx_v[pl.ds(j * 16, 16)], 1)
        wcp = pltpu.async_copy(wemb_hbm.at[widx2_v], wrows_v, sem0)
        ccps = [
            pltpu.async_copy(cemb_hbm.at[cidx2_v.at[pl.ds(j * 128, 128)]],
                             crows_v.at[pl.ds(j * 128, 128)], sem1)
            for j in range(CROWS // 128)]
        wcp.wait()
        for cp in ccps:
            cp.wait()

        for sub in range(NSUB):
            brow = lanes + sub * 16
            wbase = lax.shift_left(
                jnp.bitwise_and(plsc.load_gather(widx_v, [brow]), 1), 6)
            ridx = [brow + (k * CHUNK + sub * 16) for k in range(K)]
            cbase = [lax.shift_left(
                jnp.bitwise_and(plsc.load_gather(cidx_v, [r]), 1), 6)
                for r in ridx]
            zeros = tuple(jnp.zeros((16,), jnp.float32) for _ in range(K))

            @plsc.parallel_loop(0, HID, 1, unroll=4, carry=zeros)
            def accs(h, accs_in):
                hcol = jnp.full((16,), h, jnp.int32)
                wv = plsc.load_gather(wrows_v, [brow, wbase + hcol])
                return tuple(
                    accs_in[k] + wv * plsc.load_gather(
                        crows_v, [ridx[k], cbase[k] + hcol])
                    for k in range(K))

            for k in range(K):
                res_v[k, pl.ds(i * CHUNK + sub * 16, 16)] = accs[k]
        return _

    lax.fori_loop(0, NCHUNK, chunk_body, 0)
    for k in range(K):
        pltpu.sync_copy(res_v.at[k], out_hbm.at[pl.ds(k * B + wid * BPW, BPW)])


@jax.jit
def _sc_dots(wemb2, cemb2, widx, cidx):
    mesh = plsc.VectorSubcoreMesh(core_axis_name="c", subcore_axis_name="s",
                                  num_cores=NC, num_subcores=NS)
    return pl.kernel(
        _sc_dots_body,
        out_type=jax.ShapeDtypeStruct((K * B,), jnp.float32),
        mesh=mesh,
        compiler_params=pltpu.CompilerParams(needs_layout_passes=False,
                                             use_tc_tiling_on_sc=True),
        scratch_types=[
            pltpu.VMEM((CHUNK,), jnp.int32),
            pltpu.VMEM((CROWS,), jnp.int32),
            pltpu.VMEM((CHUNK,), jnp.int32),
            pltpu.VMEM((CROWS,), jnp.int32),
            pltpu.VMEM((CHUNK, 2 * HID), jnp.float32),
            pltpu.VMEM((CROWS, 2 * HID), jnp.float32),
            pltpu.VMEM((K, BPW), jnp.float32),
            pltpu.SemaphoreType.DMA,
            pltpu.SemaphoreType.DMA,
            pltpu.SemaphoreType.DMA,
        ],
    )(wemb2, cemb2, widx, cidx)


def _loss_body(res_ref, lab_ref, out_ref):
    x = res_ref[...] * lab_ref[...]
    y = jax.nn.log_sigmoid(x)
    out_ref[0, 0] = -jnp.sum(y) / B


def _loss(res2d, lab2d):
    out = pl.pallas_call(
        _loss_body,
        out_shape=jax.ShapeDtypeStruct((1, 1), jnp.float32),
        in_specs=[pl.BlockSpec(memory_space=pltpu.VMEM),
                  pl.BlockSpec(memory_space=pltpu.VMEM)],
        out_specs=pl.BlockSpec(memory_space=pltpu.SMEM),
    )(res2d, lab2d)
    return out[0, 0]


def kernel(wrd, cntxt, labels, word_emb, context_emb):
    widx = wrd.reshape(B).astype(jnp.int32)
    cidx = cntxt.T.astype(jnp.int32)
    res = _sc_dots(word_emb.reshape(VOCAB // 2, 2 * HID),
                   context_emb.reshape(VOCAB // 2, 2 * HID), widx, cidx)
    res2d = res.reshape(K * B // 128, 128)
    lab2d = labels.T.reshape(K * B // 128, 128)
    return _loss(res2d, lab2d)


# speedup vs baseline: 1.0249x; 1.0249x over previous
import jax
import jax.numpy as jnp
from jax import lax
from jax.experimental import pallas as pl
from jax.experimental.pallas import tpu as pltpu
from jax.experimental.pallas import tpu_sc as plsc

B = 16384
K = 20
HID = 64

NC = 2
NS = 16
NW = NC * NS
BPW = B // NW
CHUNK = 64
NCHUNK = BPW // CHUNK
CROWS = CHUNK * K
NSUB = CHUNK // 16


def _sc_dots_body(wemb_hbm, cemb_hbm, widx_hbm, cidx_hbm, out_hbm,
                  widx_v, cidx_v, wrows_v, crows_v, res_v, sem0, sem1, sem2):
    wid = lax.axis_index("s") * NC + lax.axis_index("c")
    lanes = lax.iota(jnp.int32, 16)

    def chunk_body(i, _):
        base = wid * BPW + i * CHUNK
        pltpu.sync_copy(widx_hbm.at[pl.ds(base, CHUNK)], widx_v)
        idx_cps = [
            pltpu.async_copy(cidx_hbm.at[k, pl.ds(base, CHUNK)],
                             cidx_v.at[pl.ds(k * CHUNK, CHUNK)], sem2)
            for k in range(K)]
        for cp in idx_cps:
            cp.wait()
        wcp = pltpu.async_copy(wemb_hbm.at[widx_v], wrows_v, sem0)
        ccps = [
            pltpu.async_copy(cemb_hbm.at[cidx_v.at[pl.ds(j * 128, 128)]],
                             crows_v.at[pl.ds(j * 128, 128)], sem1)
            for j in range(CROWS // 128)]
        wcp.wait()
        for cp in ccps:
            cp.wait()

        for sub in range(NSUB):
            brow = lanes + sub * 16
            ridx = [brow + (k * CHUNK + sub * 16) for k in range(K)]
            zeros = tuple(jnp.zeros((16,), jnp.float32) for _ in range(K))

            @plsc.parallel_loop(0, HID, 1, unroll=4, carry=zeros)
            def accs(h, accs_in):
                hcol = jnp.full((16,), h, jnp.int32)
                wv = plsc.load_gather(wrows_v, [brow, hcol])
                return tuple(
                    accs_in[k]
                    + wv * plsc.load_gather(crows_v, [ridx[k], hcol])
                    for k in range(K))

            for k in range(K):
                res_v[k, pl.ds(i * CHUNK + sub * 16, 16)] = accs[k]
        return _

    lax.fori_loop(0, NCHUNK, chunk_body, 0)
    for k in range(K):
        pltpu.sync_copy(res_v.at[k], out_hbm.at[pl.ds(k * B + wid * BPW, BPW)])


@jax.jit
def _sc_dots(wemb, cemb, widx, cidx):
    mesh = plsc.VectorSubcoreMesh(core_axis_name="c", subcore_axis_name="s",
                                  num_cores=NC, num_subcores=NS)
    return pl.kernel(
        _sc_dots_body,
        out_type=jax.ShapeDtypeStruct((K * B,), jnp.float32),
        mesh=mesh,
        compiler_params=pltpu.CompilerParams(needs_layout_passes=False,
                                             use_tc_tiling_on_sc=False),
        scratch_types=[
            pltpu.VMEM((CHUNK,), jnp.int32),
            pltpu.VMEM((CROWS,), jnp.int32),
            pltpu.VMEM((CHUNK, HID), jnp.float32),
            pltpu.VMEM((CROWS, HID), jnp.float32),
            pltpu.VMEM((K, BPW), jnp.float32),
            pltpu.SemaphoreType.DMA,
            pltpu.SemaphoreType.DMA,
            pltpu.SemaphoreType.DMA,
        ],
    )(wemb, cemb, widx, cidx)


def _loss_body(res_ref, lab_ref, out_ref):
    x = res_ref[...] * lab_ref[...]
    y = jax.nn.log_sigmoid(x)
    out_ref[0, 0] = -jnp.sum(y) / B


def _loss(res2d, lab2d):
    out = pl.pallas_call(
        _loss_body,
        out_shape=jax.ShapeDtypeStruct((1, 1), jnp.float32),
        in_specs=[pl.BlockSpec(memory_space=pltpu.VMEM),
                  pl.BlockSpec(memory_space=pltpu.VMEM)],
        out_specs=pl.BlockSpec(memory_space=pltpu.SMEM),
    )(res2d, lab2d)
    return out[0, 0]


def kernel(wrd, cntxt, labels, word_emb, context_emb):
    widx = wrd.reshape(B).astype(jnp.int32)
    cidx = cntxt.T.astype(jnp.int32)
    res = _sc_dots(word_emb, context_emb, widx, cidx)
    res2d = res.reshape(K * B // 128, 128)
    lab2d = labels.T.reshape(K * B // 128, 128)
    return _loss(res2d, lab2d)
